# Initial kernel scaffold; baseline (speedup 1.0000x reference)
import jax
import jax.numpy as jnp
from jax.experimental import pallas as pl


def kernel(x, weight):
    raise NotImplementedError("write your pallas kernel here")


# speedup vs baseline: 1.0950x; 1.0950x over previous
import functools

import jax
import jax.numpy as jnp
from jax import lax
from jax.experimental import pallas as pl
from jax.experimental.pallas import tpu as pltpu
from jax.experimental.pallas import tpu_sc as plsc

_DIM = 32
_NC = 2
_NS = 16
_NW = _NC * _NS

_SUB = 128
_NSUB = 8
_C = _SUB * _NSUB


@functools.lru_cache(maxsize=None)
def _make_gather(B: int):
  b_per_w = B // _NW
  nchunks = b_per_w // _C
  assert b_per_w % _C == 0 and B % _NW == 0

  mesh = plsc.VectorSubcoreMesh(core_axis_name="c", subcore_axis_name="s")

  @functools.partial(
      pl.kernel,
      out_type=jax.ShapeDtypeStruct((B, _DIM), jnp.float32),
      mesh=mesh,
      compiler_params=pltpu.CompilerParams(use_tc_tiling_on_sc=False),
      scratch_types=[
          pltpu.VMEM((_NSUB, _SUB), jnp.int32),
          pltpu.VMEM((_C, _DIM), jnp.float32),
          pltpu.SemaphoreType.DMA,
          pltpu.SemaphoreType.DMA,
      ],
  )
  def gather_kernel(idx_hbm, table_hbm, out_hbm, idx_v, rows_v, gsem, isem):
    wid = lax.axis_index("s") * _NC + lax.axis_index("c")
    base = wid * b_per_w
    row0 = base // _SUB

    def chunk(g, carry):
      irow = pl.multiple_of(row0 + g * _NSUB, 8)
      orow = pl.multiple_of(base + g * _C, 8)
      pltpu.async_copy(
          idx_hbm.at[pl.ds(irow, _NSUB)], idx_v, isem).wait()
      for j in range(_NSUB):
        pltpu.async_copy(
            table_hbm.at[idx_v.at[j]],
            rows_v.at[pl.ds(j * _SUB, _SUB)],
            gsem)
      pltpu.make_async_copy(table_hbm.at[pl.ds(0, _C)], rows_v, gsem).wait()
      pltpu.async_copy(
          rows_v, out_hbm.at[pl.ds(orow, _C)], isem).wait()
      return carry

    lax.fori_loop(0, nchunks, chunk, 0)

  return gather_kernel


def kernel(x, weight):
  B = x.shape[0] * x.shape[1]
  idx2d = x.reshape(B // _SUB, _SUB).astype(jnp.int32)
  out = _make_gather(B)(idx2d, weight)
  return out.reshape(x.shape[0], x.shape[1], _DIM)
